# baseline (speedup 1.0000x reference)
import jax
import jax.numpy as jnp
from jax import lax
from jax.experimental import pallas as pl
from jax.experimental.pallas import tpu as pltpu
from jax.experimental.pallas import tpu_sc as plsc

NN = 10000
NE = 320000
D = 128
NC, NS = 2, 16
NW = NC * NS
EPW = NE // NW
CH = 80
NCHK = EPW // CH
CH2 = 128
NCHK2 = 80
EPWP = NCHK2 * CH2
NDUMP = 64
WCH = 80
NWC = NN // WCH
TRIPS = -(-NWC // NS)

_mesh = plsc.VectorSubcoreMesh(
    core_axis_name="c", subcore_axis_name="s", num_cores=NC, num_subcores=NS)


def _deg_body(dst_hbm, out_hbm, dstv, onesv, zv, dacc):
    c = lax.axis_index("c")
    s = lax.axis_index("s")
    w = s * NC + c

    def fill_ones(i, carry):
        onesv[i, :] = jnp.ones((16,), jnp.float32)
        return carry

    lax.fori_loop(0, CH, fill_ones, 0)

    def fill_z(i, carry):
        zv[i, :] = jnp.zeros((16,), jnp.float32)
        return carry

    lax.fori_loop(0, WCH, fill_z, 0)

    def zchunk(k, carry):
        t = k * NS + s

        @pl.when(t < NWC)
        def _():
            pltpu.sync_copy(zv, dacc.at[pl.ds(t * WCH, WCH)])

        return carry

    lax.fori_loop(0, TRIPS, zchunk, 0)
    plsc.subcore_barrier()

    pltpu.sync_copy(dst_hbm.at[w], dstv)

    def step(k, carry):
        pltpu.sync_copy(onesv, dacc.at[dstv.at[k]], add=True)
        return carry

    lax.fori_loop(0, NCHK, step, 0)
    plsc.subcore_barrier()

    def wchunk(k, carry):
        t = k * NS + s

        @pl.when(t < NWC)
        def _():
            sl = pl.ds(t * WCH, WCH)
            pltpu.sync_copy(dacc.at[sl], out_hbm.at[c, sl])

        return carry

    lax.fori_loop(0, TRIPS, wchunk, 0)


_deg_kernel = pl.kernel(
    _deg_body,
    out_type=jax.ShapeDtypeStruct((NC, NN, 16), jnp.float32),
    mesh=_mesh,
    scratch_types=[
        pltpu.VMEM((NCHK, CH), jnp.int32),
        pltpu.VMEM((CH, 16), jnp.float32),
        pltpu.VMEM((WCH, 16), jnp.float32),
        pltpu.VMEM_SHARED((NN, 16), jnp.float32),
    ],
)


def _seg_body(hs_hbm, src_hbm, dst_hbm, out_hbm, srcv, dstv, rowsv, acc, sem):
    c = lax.axis_index("c")
    s = lax.axis_index("s")
    w = s * NC + c

    def fill_z(i, carry):
        def fz2(j, carry2):
            rowsv[i, pl.ds(j * 16, 16)] = jnp.zeros((16,), jnp.float32)
            return carry2

        lax.fori_loop(0, D // 16, fz2, 0)
        return carry

    lax.fori_loop(0, WCH, fill_z, 0)

    def zchunk(k, carry):
        t = k * NS + s

        @pl.when(t < NWC)
        def _():
            pltpu.sync_copy(rowsv.at[pl.ds(0, WCH)],
                            acc.at[pl.ds(t * WCH, WCH)])

        return carry

    lax.fori_loop(0, TRIPS, zchunk, 0)
    plsc.subcore_barrier()

    pltpu.sync_copy(src_hbm.at[w], srcv)
    pltpu.sync_copy(dst_hbm.at[w], dstv)

    def step(k, carry):
        pltpu.async_copy(hs_hbm.at[srcv.at[k]], rowsv, sem).wait()
        pltpu.sync_copy(rowsv, acc.at[dstv.at[k]], add=True)
        return carry

    lax.fori_loop(0, NCHK2, step, 0)
    plsc.subcore_barrier()

    def wchunk(k, carry):
        t = k * NS + s

        @pl.when(t < NWC)
        def _():
            sl = pl.ds(t * WCH, WCH)
            pltpu.sync_copy(acc.at[sl], out_hbm.at[c, sl])

        return carry

    lax.fori_loop(0, TRIPS, wchunk, 0)


_seg_kernel = pl.kernel(
    _seg_body,
    out_type=jax.ShapeDtypeStruct((NC, NN, D), jnp.float32),
    mesh=_mesh,
    scratch_types=[
        pltpu.VMEM((NCHK2, CH2), jnp.int32),
        pltpu.VMEM((NCHK2, CH2), jnp.int32),
        pltpu.VMEM((CH2, D), jnp.float32),
        pltpu.VMEM_SHARED((NN + NDUMP, D), jnp.float32),
        pltpu.SemaphoreType.DMA,
    ],
)

BLK = 2000
GRID = NN // BLK


def _dis_block(deg_ref):
    d = deg_ref[0, :, 0:1] + deg_ref[1, :, 0:1] + 1.0
    return lax.rsqrt(d)


def _tc_in_body(x_ref, deg_ref, win_ref, bin_ref, wg0_ref, hs_ref):
    h = jnp.maximum(
        jnp.dot(x_ref[...], win_ref[...], preferred_element_type=jnp.float32)
        + bin_ref[...], 0.0)
    hs_ref[...] = jnp.dot(
        h, wg0_ref[...], preferred_element_type=jnp.float32) * _dis_block(deg_ref)


_tc_in = pl.pallas_call(
    _tc_in_body,
    grid=(GRID,),
    in_specs=[
        pl.BlockSpec((BLK, D), lambda i: (i, 0)),
        pl.BlockSpec((NC, BLK, 16), lambda i: (0, i, 0)),
        pl.BlockSpec((D, D), lambda i: (0, 0)),
        pl.BlockSpec((1, D), lambda i: (0, 0)),
        pl.BlockSpec((D, D), lambda i: (0, 0)),
    ],
    out_specs=pl.BlockSpec((BLK, D), lambda i: (i, 0)),
    out_shape=jax.ShapeDtypeStruct((NN, D), jnp.float32),
)


def _tc_mid_body(s_ref, hs_ref, deg_ref, b_ref, w_ref, o_ref):
    dis = _dis_block(deg_ref)
    h = jnp.maximum(
        dis * (s_ref[0] + s_ref[1] + hs_ref[...]) + b_ref[...], 0.0)
    o_ref[...] = jnp.dot(
        h, w_ref[...], preferred_element_type=jnp.float32) * dis


_tc_mid = pl.pallas_call(
    _tc_mid_body,
    grid=(GRID,),
    in_specs=[
        pl.BlockSpec((NC, BLK, D), lambda i: (0, i, 0)),
        pl.BlockSpec((BLK, D), lambda i: (i, 0)),
        pl.BlockSpec((NC, BLK, 16), lambda i: (0, i, 0)),
        pl.BlockSpec((1, D), lambda i: (0, 0)),
        pl.BlockSpec((D, D), lambda i: (0, 0)),
    ],
    out_specs=pl.BlockSpec((BLK, D), lambda i: (i, 0)),
    out_shape=jax.ShapeDtypeStruct((NN, D), jnp.float32),
)


def _tc_fin_body(s_ref, hs_ref, deg_ref, bg_ref, wo1_ref, bo1_ref, wo2_ref,
                 bo2_ref, o_ref, acc_ref):
    i = pl.program_id(0)
    dis = _dis_block(deg_ref)
    h = jnp.maximum(
        dis * (s_ref[0] + s_ref[1] + hs_ref[...]) + bg_ref[...], 0.0)
    part = jnp.sum(h, axis=0, keepdims=True)

    @pl.when(i == 0)
    def _():
        acc_ref[...] = part

    @pl.when(i > 0)
    def _():
        acc_ref[...] += part

    @pl.when(i == GRID - 1)
    def _():
        g = acc_ref[...] * (1.0 / NN)
        o = jnp.maximum(
            jnp.dot(g, wo1_ref[...], preferred_element_type=jnp.float32)
            + bo1_ref[...], 0.0)
        o_ref[...] = jnp.dot(
            o, wo2_ref[...], preferred_element_type=jnp.float32) + bo2_ref[...]


_tc_fin = pl.pallas_call(
    _tc_fin_body,
    grid=(GRID,),
    in_specs=[
        pl.BlockSpec((NC, BLK, D), lambda i: (0, i, 0)),
        pl.BlockSpec((BLK, D), lambda i: (i, 0)),
        pl.BlockSpec((NC, BLK, 16), lambda i: (0, i, 0)),
        pl.BlockSpec((1, D), lambda i: (0, 0)),
        pl.BlockSpec((D, D), lambda i: (0, 0)),
        pl.BlockSpec((1, D), lambda i: (0, 0)),
        pl.BlockSpec((D, D), lambda i: (0, 0)),
        pl.BlockSpec((1, D), lambda i: (0, 0)),
    ],
    out_specs=pl.BlockSpec((1, D), lambda i: (0, 0)),
    out_shape=jax.ShapeDtypeStruct((1, D), jnp.float32),
    scratch_shapes=[pltpu.VMEM((1, D), jnp.float32)],
)


def kernel(x, edge_index, W_in, b_in, W_g0, b_g0, W_g1, b_g1, W_g2, b_g2,
           W_o1, b_o1, W_o2, b_o2):
    srcw = edge_index[0].reshape(NW, EPW)
    dstw = edge_index[1].reshape(NW, EPW)
    pad = EPWP - EPW
    src = jnp.pad(srcw, ((0, 0), (0, pad))).reshape(NW, NCHK2, CH2)
    dump = NN + (jnp.arange(pad, dtype=jnp.int32) % NDUMP)
    dst = jnp.concatenate(
        [dstw, jnp.broadcast_to(dump, (NW, pad))],
        axis=1).reshape(NW, NCHK2, CH2)
    dst3 = edge_index[1].reshape(NW, NCHK, CH)
    deg16 = _deg_kernel(dst3)
    hs0 = _tc_in(x, deg16, W_in, b_in.reshape(1, D), W_g0)
    s0 = _seg_kernel(hs0, src, dst)
    hs1 = _tc_mid(s0, hs0, deg16, b_g0.reshape(1, D), W_g1)
    s1 = _seg_kernel(hs1, src, dst)
    hs2 = _tc_mid(s1, hs1, deg16, b_g1.reshape(1, D), W_g2)
    s2 = _seg_kernel(hs2, src, dst)
    return _tc_fin(s2, hs2, deg16, b_g2.reshape(1, D), W_o1,
                   b_o1.reshape(1, D), W_o2, b_o2.reshape(1, D))


# speedup vs baseline: 2.3401x; 2.3401x over previous
import jax
import jax.numpy as jnp
from jax import lax
from jax.experimental import pallas as pl
from jax.experimental.pallas import tpu as pltpu
from jax.experimental.pallas import tpu_sc as plsc

NN = 10000
NE = 320000
D = 128
NC, NS = 2, 16
NW = NC * NS
EPW = NE // NW
CH = 80
NCHK = EPW // CH
CH2 = 128
NCHK2 = 80
EPWP = NCHK2 * CH2
NDUMP = 64
WCH = 80
NWC = NN // WCH
TRIPS = -(-NWC // NS)

_mesh = plsc.VectorSubcoreMesh(
    core_axis_name="c", subcore_axis_name="s", num_cores=NC, num_subcores=NS)


def _deg_body(dst_hbm, out_hbm, dstv, onesv, zv, dacc):
    c = lax.axis_index("c")
    s = lax.axis_index("s")
    w = s * NC + c

    def fill_ones(i, carry):
        onesv[i, :] = jnp.ones((16,), jnp.float32)
        return carry

    lax.fori_loop(0, CH, fill_ones, 0)

    def fill_z(i, carry):
        zv[i, :] = jnp.zeros((16,), jnp.float32)
        return carry

    lax.fori_loop(0, WCH, fill_z, 0)

    def zchunk(k, carry):
        t = k * NS + s

        @pl.when(t < NWC)
        def _():
            pltpu.sync_copy(zv, dacc.at[pl.ds(t * WCH, WCH)])

        return carry

    lax.fori_loop(0, TRIPS, zchunk, 0)
    plsc.subcore_barrier()

    pltpu.sync_copy(dst_hbm.at[w], dstv)

    def step(k, carry):
        pltpu.sync_copy(onesv, dacc.at[dstv.at[k]], add=True)
        return carry

    lax.fori_loop(0, NCHK, step, 0)
    plsc.subcore_barrier()

    def wchunk(k, carry):
        t = k * NS + s

        @pl.when(t < NWC)
        def _():
            sl = pl.ds(t * WCH, WCH)
            pltpu.sync_copy(dacc.at[sl], out_hbm.at[c, sl])

        return carry

    lax.fori_loop(0, TRIPS, wchunk, 0)


_deg_kernel = pl.kernel(
    _deg_body,
    out_type=jax.ShapeDtypeStruct((NC, NN, 16), jnp.float32),
    mesh=_mesh,
    scratch_types=[
        pltpu.VMEM((NCHK, CH), jnp.int32),
        pltpu.VMEM((CH, 16), jnp.float32),
        pltpu.VMEM((WCH, 16), jnp.float32),
        pltpu.VMEM_SHARED((NN, 16), jnp.float32),
    ],
)


def _seg_body(hs_hbm, src_hbm, dst_hbm, out_hbm, srcv, dstv, rowsv, acc, sem):
    c = lax.axis_index("c")
    s = lax.axis_index("s")
    w = s * NC + c

    def fill_z(i, carry):
        def fz2(j, carry2):
            rowsv[i, pl.ds(j * 16, 16)] = jnp.zeros((16,), jnp.float32)
            return carry2

        lax.fori_loop(0, D // 16, fz2, 0)
        return carry

    lax.fori_loop(0, WCH, fill_z, 0)

    def zchunk(k, carry):
        t = k * NS + s

        @pl.when(t < NWC)
        def _():
            pltpu.sync_copy(rowsv.at[pl.ds(0, WCH)],
                            acc.at[pl.ds(t * WCH, WCH)])

        return carry

    lax.fori_loop(0, TRIPS, zchunk, 0)
    plsc.subcore_barrier()

    pltpu.sync_copy(src_hbm.at[w], srcv)
    pltpu.sync_copy(dst_hbm.at[w], dstv)

    def step(k, carry):
        pltpu.async_copy(hs_hbm.at[srcv.at[k]], rowsv, sem).wait()
        pltpu.sync_copy(rowsv, acc.at[dstv.at[k]], add=True)
        return carry

    lax.fori_loop(0, NCHK2, step, 0)
    plsc.subcore_barrier()

    def wchunk(k, carry):
        t = k * NS + s

        @pl.when(t < NWC)
        def _():
            sl = pl.ds(t * WCH, WCH)
            pltpu.sync_copy(acc.at[sl], out_hbm.at[c, sl])

        return carry

    lax.fori_loop(0, TRIPS, wchunk, 0)


_seg_kernel = pl.kernel(
    _seg_body,
    out_type=jax.ShapeDtypeStruct((NC, NN, D), jnp.float32),
    mesh=_mesh,
    scratch_types=[
        pltpu.VMEM((NCHK2, CH2), jnp.int32),
        pltpu.VMEM((NCHK2, CH2), jnp.int32),
        pltpu.VMEM((CH2, D), jnp.float32),
        pltpu.VMEM_SHARED((NN + NDUMP, D), jnp.float32),
        pltpu.SemaphoreType.DMA,
    ],
)

BLK = 2000
GRID = NN // BLK


def _dis_block(deg_ref):
    d = deg_ref[0, :, 0:1] + deg_ref[1, :, 0:1] + 1.0
    return lax.rsqrt(d)


def _tc_in_body(x_ref, deg_ref, win_ref, bin_ref, wg0_ref, hs_ref):
    h = jnp.maximum(
        jnp.dot(x_ref[...], win_ref[...], preferred_element_type=jnp.float32)
        + bin_ref[...], 0.0)
    hs_ref[...] = jnp.dot(
        h, wg0_ref[...], preferred_element_type=jnp.float32) * _dis_block(deg_ref)


_tc_in = pl.pallas_call(
    _tc_in_body,
    grid=(GRID,),
    in_specs=[
        pl.BlockSpec((BLK, D), lambda i: (i, 0)),
        pl.BlockSpec((NC, BLK, 16), lambda i: (0, i, 0)),
        pl.BlockSpec((D, D), lambda i: (0, 0)),
        pl.BlockSpec((1, D), lambda i: (0, 0)),
        pl.BlockSpec((D, D), lambda i: (0, 0)),
    ],
    out_specs=pl.BlockSpec((BLK, D), lambda i: (i, 0)),
    out_shape=jax.ShapeDtypeStruct((NN, D), jnp.float32),
)


def _tc_mid_body(s_ref, hs_ref, deg_ref, b_ref, w_ref, o_ref):
    dis = _dis_block(deg_ref)
    h = jnp.maximum(
        dis * (s_ref[0] + s_ref[1] + hs_ref[...]) + b_ref[...], 0.0)
    o_ref[...] = jnp.dot(
        h, w_ref[...], preferred_element_type=jnp.float32) * dis


_tc_mid = pl.pallas_call(
    _tc_mid_body,
    grid=(GRID,),
    in_specs=[
        pl.BlockSpec((NC, BLK, D), lambda i: (0, i, 0)),
        pl.BlockSpec((BLK, D), lambda i: (i, 0)),
        pl.BlockSpec((NC, BLK, 16), lambda i: (0, i, 0)),
        pl.BlockSpec((1, D), lambda i: (0, 0)),
        pl.BlockSpec((D, D), lambda i: (0, 0)),
    ],
    out_specs=pl.BlockSpec((BLK, D), lambda i: (i, 0)),
    out_shape=jax.ShapeDtypeStruct((NN, D), jnp.float32),
)


def _tc_fin_body(s_ref, hs_ref, deg_ref, bg_ref, wo1_ref, bo1_ref, wo2_ref,
                 bo2_ref, o_ref, acc_ref):
    i = pl.program_id(0)
    dis = _dis_block(deg_ref)
    h = jnp.maximum(
        dis * (s_ref[0] + s_ref[1] + hs_ref[...]) + bg_ref[...], 0.0)
    part = jnp.sum(h, axis=0, keepdims=True)

    @pl.when(i == 0)
    def _():
        acc_ref[...] = part

    @pl.when(i > 0)
    def _():
        acc_ref[...] += part

    @pl.when(i == GRID - 1)
    def _():
        g = acc_ref[...] * (1.0 / NN)
        o = jnp.maximum(
            jnp.dot(g, wo1_ref[...], preferred_element_type=jnp.float32)
            + bo1_ref[...], 0.0)
        o_ref[...] = jnp.dot(
            o, wo2_ref[...], preferred_element_type=jnp.float32) + bo2_ref[...]


_tc_fin = pl.pallas_call(
    _tc_fin_body,
    grid=(GRID,),
    in_specs=[
        pl.BlockSpec((NC, BLK, D), lambda i: (0, i, 0)),
        pl.BlockSpec((BLK, D), lambda i: (i, 0)),
        pl.BlockSpec((NC, BLK, 16), lambda i: (0, i, 0)),
        pl.BlockSpec((1, D), lambda i: (0, 0)),
        pl.BlockSpec((D, D), lambda i: (0, 0)),
        pl.BlockSpec((1, D), lambda i: (0, 0)),
        pl.BlockSpec((D, D), lambda i: (0, 0)),
        pl.BlockSpec((1, D), lambda i: (0, 0)),
    ],
    out_specs=pl.BlockSpec((1, D), lambda i: (0, 0)),
    out_shape=jax.ShapeDtypeStruct((1, D), jnp.float32),
    scratch_shapes=[pltpu.VMEM((1, D), jnp.float32)],
)


def kernel(x, edge_index, W_in, b_in, W_g0, b_g0, W_g1, b_g1, W_g2, b_g2,
           W_o1, b_o1, W_o2, b_o2):
    srcw = edge_index[0].reshape(NW, EPW)
    dstw = edge_index[1].reshape(NW, EPW)
    pad = EPWP - EPW
    srcpad = jnp.arange(pad, dtype=jnp.int32) * 37 % NN
    src = jnp.concatenate(
        [srcw, jnp.broadcast_to(srcpad, (NW, pad))],
        axis=1).reshape(NW, NCHK2, CH2)
    dump = NN + (jnp.arange(pad, dtype=jnp.int32) % NDUMP)
    dst = jnp.concatenate(
        [dstw, jnp.broadcast_to(dump, (NW, pad))],
        axis=1).reshape(NW, NCHK2, CH2)
    dst3 = edge_index[1].reshape(NW, NCHK, CH)
    deg16 = _deg_kernel(dst3)
    hs0 = _tc_in(x, deg16, W_in, b_in.reshape(1, D), W_g0)
    s0 = _seg_kernel(hs0, src, dst)
    hs1 = _tc_mid(s0, hs0, deg16, b_g0.reshape(1, D), W_g1)
    s1 = _seg_kernel(hs1, src, dst)
    hs2 = _tc_mid(s1, hs1, deg16, b_g1.reshape(1, D), W_g2)
    s2 = _seg_kernel(hs2, src, dst)
    return _tc_fin(s2, hs2, deg16, b_g2.reshape(1, D), W_o1,
                   b_o1.reshape(1, D), W_o2, b_o2.reshape(1, D))
